# baseline (speedup 1.0000x reference)
import jax
import jax.numpy as jnp
from jax import lax
from jax.experimental import pallas as pl
from jax.experimental.pallas import tpu as pltpu

_BB = 16
_N = 256
_KT = 3


def _glu_c(a, n, t_in, kt, w_stk_t, b_t, a_t):
    cin = a.shape[0]
    co = w_stk_t.shape[0] // 2
    to = t_in - kt + 1
    if cin % 8 == 0:
        a_stk = jnp.concatenate([a[:, k * n:(k + to) * n] for k in range(kt)],
                                axis=0)
        acc = jnp.dot(w_stk_t, a_stk, preferred_element_type=jnp.float32)
    else:
        acc = jnp.dot(w_stk_t[:, 0:cin], a[:, 0:to * n],
                      preferred_element_type=jnp.float32)
        for k in range(1, kt):
            acc = acc + jnp.dot(w_stk_t[:, k * cin:(k + 1) * cin],
                                a[:, k * n:(k + to) * n],
                                preferred_element_type=jnp.float32)
    acc = acc + b_t
    res = jnp.dot(a_t, a[:, (kt - 1) * n:(kt - 1 + to) * n],
                  preferred_element_type=jnp.float32)
    return (acc[:co] + res) * jax.nn.sigmoid(acc[co:])


def _cheb_multi(hs, l_cat, th_t, bs_t, n, to1, f_ref):
    ks = l_cat.shape[1] // n
    outs = [[] for _ in hs]
    for t in range(to1):
        for bi, h in enumerate(hs):
            g = jnp.dot(h[:, t * n:(t + 1) * n].astype(jnp.bfloat16), l_cat,
                        preferred_element_type=jnp.float32)
            g_stk = jnp.concatenate(
                [g[:, k * n:(k + 1) * n] for k in range(ks)], axis=0)
            z = jax.nn.relu(jnp.dot(th_t, g_stk.astype(jnp.bfloat16),
                                    preferred_element_type=jnp.float32) + bs_t)
            outs[bi].append(z)
    x2 = [jnp.concatenate(o, axis=1) for o in outs]
    for bi in range(len(hs)):
        f_ref[bi] = x2[bi]
    return x2


def _seg_bsum(row, n, to):
    r = row.reshape(to, n)
    s = jnp.sum(r, axis=1, keepdims=True)
    return jnp.broadcast_to(s, (to, n)).reshape(1, to * n)


def _ln_c(x, g_t, b_t, n, to):
    c = x.shape[0]
    inv = 1.0 / float(c * n)
    s1 = jnp.sum(x, axis=0, keepdims=True)
    mu = _seg_bsum(s1, n, to) * inv
    xc = x - mu
    s2 = jnp.sum(xc * xc, axis=0, keepdims=True)
    var = _seg_bsum(s2, n, to) * inv
    rs = lax.rsqrt(var + 1e-5)
    g_full = jnp.concatenate([g_t] * to, axis=1)
    b_full = jnp.concatenate([b_t] * to, axis=1)
    return xc * rs * g_full + b_full


def _stgcn_kernel(x_ref, lcat_ref,
                  w1a_ref, b1a_ref, a1a_ref, th0_ref, bs0_ref,
                  w2a_ref, b2a_ref, a2a_ref, g0_ref, be0_ref,
                  w1b_ref, b1b_ref, a1b_ref, th1_ref, bs1_ref,
                  w2b_ref, b2b_ref, a2b_ref, g1_ref, be1_ref,
                  wt_ref, bt_ref, at_ref, gh_ref, beh_ref,
                  ws_ref, bso_ref, wf_ref, bfc_ref,
                  o_ref, f0_ref, f1_ref):
    n = _N
    kt = _KT
    l_cat = lcat_ref[...]
    t_in = x_ref.shape[2] // n
    to1 = t_in - kt + 1
    to2 = to1 - kt + 1
    to3 = to2 - kt + 1
    to4 = to3 - kt + 1
    bb = range(_BB)

    h0 = [_glu_c(x_ref[bi], n, t_in, kt, w1a_ref[...], b1a_ref[...],
                 a1a_ref[...]) for bi in bb]
    x2 = _cheb_multi(h0, l_cat, th0_ref[...], bs0_ref[...], n, to1, f0_ref)
    h1 = [_glu_c(x2[bi], n, to1, kt, w2a_ref[...], b2a_ref[...],
                 a2a_ref[...]) for bi in bb]
    a2 = [_ln_c(h1[bi], g0_ref[...], be0_ref[...], n, to2) for bi in bb]

    h2 = [_glu_c(a2[bi], n, to2, kt, w1b_ref[...], b1b_ref[...],
                 a1b_ref[...]) for bi in bb]
    y2 = _cheb_multi(h2, l_cat, th1_ref[...], bs1_ref[...], n, to3, f1_ref)
    h3 = [_glu_c(y2[bi], n, to3, kt, w2b_ref[...], b2b_ref[...],
                 a2b_ref[...]) for bi in bb]
    a4 = [_ln_c(h3[bi], g1_ref[...], be1_ref[...], n, to4) for bi in bb]

    ko = to4
    for bi in bb:
        z1 = _glu_c(a4[bi], n, ko, ko, wt_ref[...], bt_ref[...], at_ref[...])
        mu = jnp.mean(z1)
        var = jnp.mean((z1 - mu) ** 2)
        z2 = (z1 - mu) * lax.rsqrt(var + 1e-5) * gh_ref[...] + beh_ref[...]
        s = jax.nn.sigmoid(jnp.dot(ws_ref[...], z2,
                                   preferred_element_type=jnp.float32)
                           + bso_ref[...])
        o_ref[bi] = jnp.dot(wf_ref[...], s,
                            preferred_element_type=jnp.float32) + bfc_ref[...]


def _tap_stack_t(w):
    kt, cin, co2 = w.shape
    return jnp.transpose(w, (2, 0, 1)).reshape(co2, kt * cin)


def kernel(x, llist, b0_w_t1, b0_b_t1, b0_a_t1, b0_theta, b0_b_s, b0_w_t2,
           b0_b_t2, b0_a_t2, b0_ln_g, b0_ln_b, b1_w_t1, b1_b_t1, b1_a_t1,
           b1_theta, b1_b_s, b1_w_t2, b1_b_t2, b1_a_t2, b1_ln_g, b1_ln_b,
           out_w_t, out_b_t, out_a_t, out_ln_g, out_ln_b, out_w_s, out_b_s,
           out_w_fc, out_b_fc):
    bsz, t_in, n, cin = x.shape
    kt = _KT
    to1 = t_in - kt + 1
    to2 = to1 - kt + 1
    to3 = to2 - kt + 1
    to4 = to3 - kt + 1
    cs0 = b0_theta.shape[2]
    cs1 = b1_theta.shape[2]
    c0 = b0_w_t2.shape[2] // 2
    c1 = b1_w_t2.shape[2] // 2

    x_t = jnp.transpose(x, (0, 3, 1, 2)).reshape(bsz, cin, t_in * n)
    l_cat = jnp.transpose(llist, (2, 0, 1)).reshape(n, llist.shape[0] * n)
    bf = lambda w: w.astype(jnp.bfloat16)
    ops = (bf(l_cat),
           _tap_stack_t(b0_w_t1), b0_b_t1.T, b0_a_t1.T,
           bf(_tap_stack_t(b0_theta)), b0_b_s.T,
           _tap_stack_t(b0_w_t2), b0_b_t2.T, b0_a_t2.T,
           b0_ln_g.T, b0_ln_b.T,
           _tap_stack_t(b1_w_t1), b1_b_t1.T, b1_a_t1.T,
           bf(_tap_stack_t(b1_theta)), b1_b_s.T,
           _tap_stack_t(b1_w_t2), b1_b_t2.T, b1_a_t2.T,
           b1_ln_g.T, b1_ln_b.T,
           _tap_stack_t(out_w_t), out_b_t.T, out_a_t.T,
           out_ln_g.T, out_ln_b.T,
           out_w_s.T, out_b_s.T, out_w_fc.T, out_b_fc.T)

    full = lambda shape: pl.BlockSpec(shape, lambda b, _s=shape: (0,) * len(_s))
    o, f0c, f1c = pl.pallas_call(
        _stgcn_kernel,
        out_shape=(jax.ShapeDtypeStruct((bsz, 1, n), jnp.float32),
                   jax.ShapeDtypeStruct((bsz, cs0, to1 * n), jnp.float32),
                   jax.ShapeDtypeStruct((bsz, cs1, to3 * n), jnp.float32)),
        grid=(bsz // _BB,),
        in_specs=[pl.BlockSpec((_BB, cin, t_in * n), lambda b: (b, 0, 0))] +
                 [full(w.shape) for w in ops],
        out_specs=(pl.BlockSpec((_BB, 1, n), lambda b: (b, 0, 0)),
                   pl.BlockSpec((_BB, cs0, to1 * n), lambda b: (b, 0, 0)),
                   pl.BlockSpec((_BB, cs1, to3 * n), lambda b: (b, 0, 0))),
        compiler_params=pltpu.CompilerParams(
            dimension_semantics=("parallel",)),
    )(x_t, *ops)
    f0 = jnp.transpose(f0c.reshape(bsz, cs0, to1, n), (0, 2, 3, 1))
    f1 = jnp.transpose(f1c.reshape(bsz, cs1, to3, n), (0, 2, 3, 1))
    return o.reshape(bsz, 1, n, 1), [f0, f1]


# speedup vs baseline: 1.1125x; 1.1125x over previous
import jax
import jax.numpy as jnp
from jax import lax
from jax.experimental import pallas as pl
from jax.experimental.pallas import tpu as pltpu

_BB = 8
_N = 256
_KT = 3


def _glu_c(a, n, t_in, kt, w_stk_t, b_t, a_t):
    cin = a.shape[0]
    co = w_stk_t.shape[0] // 2
    to = t_in - kt + 1
    if cin % 8 == 0:
        a_stk = jnp.concatenate([a[:, k * n:(k + to) * n] for k in range(kt)],
                                axis=0)
        acc = jnp.dot(w_stk_t, a_stk, preferred_element_type=jnp.float32)
    else:
        acc = jnp.dot(w_stk_t[:, 0:cin], a[:, 0:to * n],
                      preferred_element_type=jnp.float32)
        for k in range(1, kt):
            acc = acc + jnp.dot(w_stk_t[:, k * cin:(k + 1) * cin],
                                a[:, k * n:(k + to) * n],
                                preferred_element_type=jnp.float32)
    acc = acc + b_t
    res = jnp.dot(a_t, a[:, (kt - 1) * n:(kt - 1 + to) * n],
                  preferred_element_type=jnp.float32)
    return (acc[:co] + res) * jax.nn.sigmoid(acc[co:])


def _cheb_multi(hs, l_cat, th_t, bs_t, n, to1, f_ref):
    ks = l_cat.shape[1] // n
    outs = [[] for _ in hs]
    for t in range(to1):
        for bi, h in enumerate(hs):
            g = jnp.dot(h[:, t * n:(t + 1) * n].astype(jnp.bfloat16), l_cat,
                        preferred_element_type=jnp.float32)
            g_stk = jnp.concatenate(
                [g[:, k * n:(k + 1) * n] for k in range(ks)], axis=0)
            z = jax.nn.relu(jnp.dot(th_t, g_stk.astype(jnp.bfloat16),
                                    preferred_element_type=jnp.float32) + bs_t)
            outs[bi].append(z)
    x2 = [jnp.concatenate(o, axis=1) for o in outs]
    for bi in range(len(hs)):
        f_ref[bi] = x2[bi]
    return x2


def _seg_bsum(row, n, to):
    r = row.reshape(to, n)
    s = jnp.sum(r, axis=1, keepdims=True)
    return jnp.broadcast_to(s, (to, n)).reshape(1, to * n)


def _ln_c(x, g_t, b_t, n, to):
    c = x.shape[0]
    inv = 1.0 / float(c * n)
    s1 = jnp.sum(x, axis=0, keepdims=True)
    mu = _seg_bsum(s1, n, to) * inv
    xc = x - mu
    s2 = jnp.sum(xc * xc, axis=0, keepdims=True)
    var = _seg_bsum(s2, n, to) * inv
    rs = lax.rsqrt(var + 1e-5)
    g_full = jnp.concatenate([g_t] * to, axis=1)
    b_full = jnp.concatenate([b_t] * to, axis=1)
    return xc * rs * g_full + b_full


def _stgcn_kernel(x_ref, lcat_ref,
                  w1a_ref, b1a_ref, a1a_ref, th0_ref, bs0_ref,
                  w2a_ref, b2a_ref, a2a_ref, g0_ref, be0_ref,
                  w1b_ref, b1b_ref, a1b_ref, th1_ref, bs1_ref,
                  w2b_ref, b2b_ref, a2b_ref, g1_ref, be1_ref,
                  wt_ref, bt_ref, at_ref, gh_ref, beh_ref,
                  ws_ref, bso_ref, wf_ref, bfc_ref,
                  o_ref, f0_ref, f1_ref):
    n = _N
    kt = _KT
    l_cat = lcat_ref[...]
    t_in = x_ref.shape[2] // n
    to1 = t_in - kt + 1
    to2 = to1 - kt + 1
    to3 = to2 - kt + 1
    to4 = to3 - kt + 1
    bb = range(_BB)

    h0 = [_glu_c(x_ref[bi], n, t_in, kt, w1a_ref[...], b1a_ref[...],
                 a1a_ref[...]) for bi in bb]
    x2 = _cheb_multi(h0, l_cat, th0_ref[...], bs0_ref[...], n, to1, f0_ref)
    h1 = [_glu_c(x2[bi], n, to1, kt, w2a_ref[...], b2a_ref[...],
                 a2a_ref[...]) for bi in bb]
    a2 = [_ln_c(h1[bi], g0_ref[...], be0_ref[...], n, to2) for bi in bb]

    h2 = [_glu_c(a2[bi], n, to2, kt, w1b_ref[...], b1b_ref[...],
                 a1b_ref[...]) for bi in bb]
    y2 = _cheb_multi(h2, l_cat, th1_ref[...], bs1_ref[...], n, to3, f1_ref)
    h3 = [_glu_c(y2[bi], n, to3, kt, w2b_ref[...], b2b_ref[...],
                 a2b_ref[...]) for bi in bb]
    a4 = [_ln_c(h3[bi], g1_ref[...], be1_ref[...], n, to4) for bi in bb]

    ko = to4
    for bi in bb:
        z1 = _glu_c(a4[bi], n, ko, ko, wt_ref[...], bt_ref[...], at_ref[...])
        mu = jnp.mean(z1)
        var = jnp.mean((z1 - mu) ** 2)
        z2 = (z1 - mu) * lax.rsqrt(var + 1e-5) * gh_ref[...] + beh_ref[...]
        s = jax.nn.sigmoid(jnp.dot(ws_ref[...], z2,
                                   preferred_element_type=jnp.float32)
                           + bso_ref[...])
        o_ref[bi] = jnp.dot(wf_ref[...], s,
                            preferred_element_type=jnp.float32) + bfc_ref[...]


def _tap_stack_t(w):
    kt, cin, co2 = w.shape
    return jnp.transpose(w, (2, 0, 1)).reshape(co2, kt * cin)


def kernel(x, llist, b0_w_t1, b0_b_t1, b0_a_t1, b0_theta, b0_b_s, b0_w_t2,
           b0_b_t2, b0_a_t2, b0_ln_g, b0_ln_b, b1_w_t1, b1_b_t1, b1_a_t1,
           b1_theta, b1_b_s, b1_w_t2, b1_b_t2, b1_a_t2, b1_ln_g, b1_ln_b,
           out_w_t, out_b_t, out_a_t, out_ln_g, out_ln_b, out_w_s, out_b_s,
           out_w_fc, out_b_fc):
    bsz, t_in, n, cin = x.shape
    kt = _KT
    to1 = t_in - kt + 1
    to2 = to1 - kt + 1
    to3 = to2 - kt + 1
    to4 = to3 - kt + 1
    cs0 = b0_theta.shape[2]
    cs1 = b1_theta.shape[2]
    c0 = b0_w_t2.shape[2] // 2
    c1 = b1_w_t2.shape[2] // 2

    x_t = jnp.transpose(x, (0, 3, 1, 2)).reshape(bsz, cin, t_in * n)
    l_cat = jnp.transpose(llist, (2, 0, 1)).reshape(n, llist.shape[0] * n)
    bf = lambda w: w.astype(jnp.bfloat16)
    ops = (bf(l_cat),
           _tap_stack_t(b0_w_t1), b0_b_t1.T, b0_a_t1.T,
           bf(_tap_stack_t(b0_theta)), b0_b_s.T,
           _tap_stack_t(b0_w_t2), b0_b_t2.T, b0_a_t2.T,
           b0_ln_g.T, b0_ln_b.T,
           _tap_stack_t(b1_w_t1), b1_b_t1.T, b1_a_t1.T,
           bf(_tap_stack_t(b1_theta)), b1_b_s.T,
           _tap_stack_t(b1_w_t2), b1_b_t2.T, b1_a_t2.T,
           b1_ln_g.T, b1_ln_b.T,
           _tap_stack_t(out_w_t), out_b_t.T, out_a_t.T,
           out_ln_g.T, out_ln_b.T,
           out_w_s.T, out_b_s.T, out_w_fc.T, out_b_fc.T)

    full = lambda shape: pl.BlockSpec(shape, lambda b, _s=shape: (0,) * len(_s))
    o, f0c, f1c = pl.pallas_call(
        _stgcn_kernel,
        out_shape=(jax.ShapeDtypeStruct((bsz, 1, n), jnp.float32),
                   jax.ShapeDtypeStruct((bsz, cs0, to1 * n), jnp.float32),
                   jax.ShapeDtypeStruct((bsz, cs1, to3 * n), jnp.float32)),
        grid=(bsz // _BB,),
        in_specs=[pl.BlockSpec((_BB, cin, t_in * n), lambda b: (b, 0, 0))] +
                 [full(w.shape) for w in ops],
        out_specs=(pl.BlockSpec((_BB, 1, n), lambda b: (b, 0, 0)),
                   pl.BlockSpec((_BB, cs0, to1 * n), lambda b: (b, 0, 0)),
                   pl.BlockSpec((_BB, cs1, to3 * n), lambda b: (b, 0, 0))),
        compiler_params=pltpu.CompilerParams(
            dimension_semantics=("parallel",)),
    )(x_t, *ops)
    f0 = jnp.transpose(f0c.reshape(bsz, cs0, to1, n), (0, 2, 3, 1))
    f1 = jnp.transpose(f1c.reshape(bsz, cs1, to3, n), (0, 2, 3, 1))
    return o.reshape(bsz, 1, n, 1), [f0, f1]
